# baseline (speedup 1.0000x reference)
import jax
import jax.numpy as jnp
import numpy as np
from jax.experimental import pallas as pl
from jax.experimental.pallas import tpu as pltpu
try:
    from jax.experimental.shard_map import shard_map
except ImportError:
    from jax.sharding import shard_map

_H = 16
_C = 64
_DH = 64
_G = 4
_GW = _G * _DH
_NG = _H // _G
_TT = 512
_SCALE = 0.125
_BF = jnp.bfloat16


def _proj_kernel(x_ref, wq_ref, wk_ref, wv_ref, q_ref, k_ref, v_ref):
    xt = x_ref[...]
    q_ref[...] = jnp.dot(
        xt, wq_ref[...], preferred_element_type=jnp.float32).astype(_BF)
    v_ref[...] = jnp.dot(
        xt, wv_ref[...], preferred_element_type=jnp.float32).astype(_BF)
    k = jnp.dot(xt, wk_ref[...], preferred_element_type=jnp.float32)
    kk = k * k
    parts = []
    for h in range(_H):
        ch = slice(h * _DH, (h + 1) * _DH)
        s = jnp.sum(kk[:, ch], axis=1, keepdims=True)
        parts.append(k[:, ch] / jnp.maximum(jnp.sqrt(s), 1e-12))
    k_ref[...] = jnp.concatenate(parts, axis=1).astype(_BF)


def _dotT(a, b):
    return jax.lax.dot_general(a, b, (((1,), (1,)), ((), ())),
                               preferred_element_type=jnp.float32)


def _dotTa(a, b):
    return jax.lax.dot_general(a, b, (((0,), (0,)), ((), ())),
                               preferred_element_type=jnp.float32)


def _dot(a, b):
    return jnp.dot(a, b, preferred_element_type=jnp.float32)


def _core_kernel(ab_ref, q_ref, k_ref, v_ref, o_ref, m_scr):
    C, GW = _C, _GW

    rowg = jax.lax.broadcasted_iota(jnp.int32, (GW, GW), 0)
    colg = jax.lax.broadcasted_iota(jnp.int32, (GW, GW), 1)
    bdmask = (rowg // C) == (colg // C)
    umask = bdmask & ((colg % C) > (rowg % C))
    eye = jnp.where(rowg == colg, 1.0, 0.0).astype(jnp.float32)
    rows_s = jax.lax.broadcasted_iota(jnp.int32, (C, GW), 0)
    cols_s = jax.lax.broadcasted_iota(jnp.int32, (C, GW), 1)
    smask = (cols_s % C) > rows_s

    absig = jax.nn.sigmoid(ab_ref[...])

    @pl.when(pl.program_id(1) == 0)
    def _():
        m_scr[...] = jnp.zeros_like(m_scr)

    def chunk_body(n, carry):
        r0 = pl.multiple_of(n * C, C)
        rows = pl.ds(r0, C)
        for g in range(_NG):
            cols = slice(g * GW, (g + 1) * GW)
            avec = absig[0:1, cols] * 0.5
            bvec = absig[1:2, cols] * 2.0

            q = q_ref[rows, cols]
            kn = k_ref[rows, cols]
            v = v_ref[rows, cols]

            kn4 = jnp.concatenate([kn, kn, kn, kn], axis=0)
            Kbd = jnp.where(bdmask, kn4, jnp.zeros((), _BF))
            v4 = jnp.concatenate([v, v, v, v], axis=0)
            Vbd = jnp.where(bdmask, v4, jnp.zeros((), _BF))

            M = m_scr[g]
            Mb = M.astype(_BF)

            scores = _dotT(q, Kbd) * _SCALE
            scores = jnp.where(smask, -1e30, scores)
            attn_parts = []
            for j in range(_G):
                cj = slice(j * _DH, (j + 1) * _DH)
                sc = scores[:, cj]
                m = jnp.max(sc, axis=1, keepdims=True)
                e = jnp.exp(sc - m)
                attn_parts.append(e / jnp.sum(e, axis=1, keepdims=True))
            attn = jnp.concatenate(attn_parts, axis=1).astype(_BF)

            attn_out = _dot(attn, Vbd)
            mem_out = _dot(q, Mb)
            o_ref[rows, cols] = (attn_out + 0.1 * mem_out).astype(_BF)

            G = _dotT(Kbd, Kbd)
            aU = jnp.where(umask, G, 0.0) * avec
            aU_b = aU.astype(_BF)
            inv_f = eye - aU
            npow = aU_b
            for _ in range(5):
                npow = _dot(npow, npow).astype(_BF)
                inv_f = inv_f + _dot(inv_f.astype(_BF), npow)
            R = _dot(inv_f.astype(_BF), Kbd)
            Ra = (R * avec).astype(_BF)
            Rb = (R * bvec).astype(_BF)
            KtRa = _dotTa(Kbd, Ra).astype(_BF)
            T1 = _dot(Mb, KtRa)
            T2 = _dotTa(Vbd, Rb)
            m_scr[g] = M - T1 + T2
        return carry

    jax.lax.fori_loop(0, _TT // C, chunk_body, 0)


def _oproj_kernel(y_ref, w_ref, o_ref):
    o_ref[...] = jnp.dot(y_ref[...], w_ref[...],
                         preferred_element_type=jnp.float32)


def _impl(x, Wqt, Wkt, Wvt, Wot, ab):
    B, T, D = x.shape
    x2 = x.reshape(B * T, D)

    RT = 512
    nrt = (B * T) // RT
    q2, k2, v2 = pl.pallas_call(
        _proj_kernel,
        grid=(nrt,),
        in_specs=[
            pl.BlockSpec((RT, D), lambda i: (i, 0)),
            pl.BlockSpec((D, D), lambda i: (0, 0)),
            pl.BlockSpec((D, D), lambda i: (0, 0)),
            pl.BlockSpec((D, D), lambda i: (0, 0)),
        ],
        out_specs=[
            pl.BlockSpec((RT, D), lambda i: (i, 0)),
            pl.BlockSpec((RT, D), lambda i: (i, 0)),
            pl.BlockSpec((RT, D), lambda i: (i, 0)),
        ],
        out_shape=[jax.ShapeDtypeStruct((B * T, D), _BF)] * 3,
        compiler_params=pltpu.CompilerParams(
            dimension_semantics=("parallel",),
            vmem_limit_bytes=52 * 1024 * 1024,
        ),
        name="titans_qkv_proj",
    )(x2, Wqt, Wkt, Wvt)

    ntt = T // _TT

    def _cidx(b, t):
        return (b * ntt + t, 0)

    y2 = pl.pallas_call(
        _core_kernel,
        grid=(B, ntt),
        in_specs=[
            pl.BlockSpec((2, D), lambda b, t: (0, 0)),
            pl.BlockSpec((_TT, D), _cidx),
            pl.BlockSpec((_TT, D), _cidx),
            pl.BlockSpec((_TT, D), _cidx),
        ],
        out_specs=pl.BlockSpec((_TT, D), _cidx),
        out_shape=jax.ShapeDtypeStruct((B * T, D), _BF),
        scratch_shapes=[pltpu.VMEM((_NG, _GW, _GW), jnp.float32)],
        compiler_params=pltpu.CompilerParams(
            dimension_semantics=("parallel", "arbitrary"),
            vmem_limit_bytes=52 * 1024 * 1024,
        ),
        name="titans_core",
    )(ab, q2, k2, v2)

    out2 = pl.pallas_call(
        _oproj_kernel,
        grid=(nrt,),
        in_specs=[
            pl.BlockSpec((RT, D), lambda i: (i, 0)),
            pl.BlockSpec((D, D), lambda i: (0, 0)),
        ],
        out_specs=pl.BlockSpec((RT, D), lambda i: (i, 0)),
        out_shape=jax.ShapeDtypeStruct((B * T, D), jnp.float32),
        compiler_params=pltpu.CompilerParams(
            dimension_semantics=("parallel",),
            vmem_limit_bytes=52 * 1024 * 1024,
        ),
        name="titans_out_proj",
    )(y2, Wot)

    return out2.reshape(B, T, D)


def kernel(x, Wq, Wk, Wv, Wo, alpha_raw, beta_raw):
    B, T, D = x.shape
    H = _H
    xb = x.astype(_BF)
    Wqt = Wq.T.astype(_BF)
    Wkt = Wk.T.astype(_BF)
    Wvt = Wv.T.astype(_BF)
    Wot = Wo.T.astype(_BF)
    ab = jnp.repeat(
        jnp.concatenate([alpha_raw.reshape(1, H), beta_raw.reshape(1, H)],
                        axis=0), D // H, axis=1)

    devs = jax.devices()
    if len(devs) >= 2 and B % 2 == 0:
        mesh = jax.sharding.Mesh(np.array(devs[:2]), ("b",))
        P = jax.sharding.PartitionSpec
        f = shard_map(
            _impl, mesh=mesh,
            in_specs=(P("b"), P(), P(), P(), P(), P()),
            out_specs=P("b"), check_rep=False)
        return f(xb, Wqt, Wkt, Wvt, Wot, ab)
    return _impl(xb, Wqt, Wkt, Wvt, Wot, ab)


# speedup vs baseline: 1.0074x; 1.0074x over previous
import jax
import jax.numpy as jnp
import numpy as np
from jax.experimental import pallas as pl
from jax.experimental.pallas import tpu as pltpu
try:
    from jax.experimental.shard_map import shard_map
except ImportError:
    from jax.sharding import shard_map

_H = 16
_C = 64
_DH = 64
_G = 4
_GW = _G * _DH
_NG = _H // _G
_TT = 512
_SCALE = 0.125
_BF = jnp.bfloat16


def _proj_kernel(x_ref, wq_ref, wk_ref, wv_ref, q_ref, k_ref, v_ref):
    xt = x_ref[...]
    D = _H * _DH
    q_ref[...] = jnp.dot(
        xt, wq_ref[...], preferred_element_type=jnp.float32).astype(_BF)
    v_ref[...] = jnp.dot(
        xt, wv_ref[...], preferred_element_type=jnp.float32).astype(_BF)
    k = jnp.dot(xt, wk_ref[...], preferred_element_type=jnp.float32)
    rd = jax.lax.broadcasted_iota(jnp.int32, (D, D), 0)
    cd = jax.lax.broadcasted_iota(jnp.int32, (D, D), 1)
    onesbd = jnp.where((rd // _DH) == (cd // _DH), 1.0, 0.0).astype(_BF)
    kk = (k * k).astype(_BF)
    s = jnp.dot(kk, onesbd, preferred_element_type=jnp.float32)
    kn = k * jax.lax.rsqrt(jnp.maximum(s, 1e-24))
    k_ref[...] = kn.astype(_BF)


def _dotT(a, b):
    return jax.lax.dot_general(a, b, (((1,), (1,)), ((), ())),
                               preferred_element_type=jnp.float32)


def _dotTa(a, b):
    return jax.lax.dot_general(a, b, (((0,), (0,)), ((), ())),
                               preferred_element_type=jnp.float32)


def _dot(a, b):
    return jnp.dot(a, b, preferred_element_type=jnp.float32)


def _core_kernel(ab_ref, q_ref, k_ref, v_ref, o_ref, m_scr):
    C, GW = _C, _GW

    rowg = jax.lax.broadcasted_iota(jnp.int32, (GW, GW), 0)
    colg = jax.lax.broadcasted_iota(jnp.int32, (GW, GW), 1)
    bdmask = (rowg // C) == (colg // C)
    lmask = bdmask & ((colg % C) < (rowg % C))
    rows_s = jax.lax.broadcasted_iota(jnp.int32, (C, GW), 0)
    cols_s = jax.lax.broadcasted_iota(jnp.int32, (C, GW), 1)
    smask = (cols_s % C) > rows_s
    ih = jnp.where((cols_s % C) == rows_s, 1.0, 0.0).astype(_BF)

    absig = jax.nn.sigmoid(ab_ref[...])

    @pl.when(pl.program_id(1) == 0)
    def _():
        m_scr[...] = jnp.zeros_like(m_scr)

    def chunk_body(n, carry):
        r0 = pl.multiple_of(n * C, C)
        rows = pl.ds(r0, C)

        qs, kns, vs, MTs = [], [], [], []
        for g in range(_NG):
            cols = slice(g * GW, (g + 1) * GW)
            qs.append(q_ref[rows, cols])
            kns.append(k_ref[rows, cols])
            vs.append(v_ref[rows, cols])
            MTs.append(m_scr[g])

        outs, newMTs = [], []
        for g in range(_NG):
            cols = slice(g * GW, (g + 1) * GW)
            avec = absig[0:1, cols] * 0.5
            bvec = absig[1:2, cols] * 2.0
            q, kn, v, MT = qs[g], kns[g], vs[g], MTs[g]

            kn4 = jnp.concatenate([kn, kn, kn, kn], axis=0)
            Kbd = jnp.where(bdmask, kn4, jnp.zeros((), _BF))
            v4 = jnp.concatenate([v, v, v, v], axis=0)
            Vbd = jnp.where(bdmask, v4, jnp.zeros((), _BF))
            MTb = MT.astype(_BF)
            mt4 = jnp.concatenate([MTb, MTb, MTb, MTb], axis=0)
            MTbd = jnp.where(bdmask, mt4, jnp.zeros((), _BF))

            scores = _dotT(q, Kbd) * _SCALE
            scores = jnp.where(smask, -1e30, scores)
            attn_parts = []
            for j in range(_G):
                cj = slice(j * _DH, (j + 1) * _DH)
                sc = scores[:, cj]
                m = jnp.max(sc, axis=1, keepdims=True)
                e = jnp.exp(sc - m)
                attn_parts.append(e / jnp.sum(e, axis=1, keepdims=True))
            attn = jnp.concatenate(attn_parts, axis=1).astype(_BF)

            attn_out = _dot(attn, Vbd)
            mem_out = _dotT(q, MTbd)
            outs.append((attn_out + 0.1 * mem_out).astype(_BF))

            G = _dotT(Kbd, Kbd)
            A = jnp.where(lmask, G, 0.0) * avec
            Ab = A.astype(_BF)
            KT = _dotT(ih, Kbd)
            Y = KT - _dot(KT.astype(_BF), Ab)
            Apow = Ab
            for _ in range(4):
                Apow = _dot(Apow, Apow).astype(_BF)
                Y = Y + _dot(Y.astype(_BF), Apow)
            Apow = _dot(Apow, Apow).astype(_BF)
            Y = Y + _dot(Y.astype(_BF), Apow)
            Ya = (Y * avec).astype(_BF)
            Yb = (Y * bvec).astype(_BF)
            KtRaT = _dot(Ya, Kbd).astype(_BF)
            T1T = _dot(KtRaT, MTbd)
            T2T = _dot(Yb, Vbd)
            newMTs.append(MT - T1T + T2T)

        for g in range(_NG):
            cols = slice(g * GW, (g + 1) * GW)
            o_ref[rows, cols] = outs[g]
            m_scr[g] = newMTs[g]
        return carry

    jax.lax.fori_loop(0, _TT // C, chunk_body, 0)


def _oproj_kernel(y_ref, w_ref, o_ref):
    o_ref[...] = jnp.dot(y_ref[...], w_ref[...],
                         preferred_element_type=jnp.float32)


def _impl(x, Wqt, Wkt, Wvt, Wot, ab):
    B, T, D = x.shape
    x2 = x.reshape(B * T, D)

    RT = 512
    nrt = (B * T) // RT
    q2, k2, v2 = pl.pallas_call(
        _proj_kernel,
        grid=(nrt,),
        in_specs=[
            pl.BlockSpec((RT, D), lambda i: (i, 0)),
            pl.BlockSpec((D, D), lambda i: (0, 0)),
            pl.BlockSpec((D, D), lambda i: (0, 0)),
            pl.BlockSpec((D, D), lambda i: (0, 0)),
        ],
        out_specs=[
            pl.BlockSpec((RT, D), lambda i: (i, 0)),
            pl.BlockSpec((RT, D), lambda i: (i, 0)),
            pl.BlockSpec((RT, D), lambda i: (i, 0)),
        ],
        out_shape=[jax.ShapeDtypeStruct((B * T, D), _BF)] * 3,
        compiler_params=pltpu.CompilerParams(
            dimension_semantics=("parallel",),
            vmem_limit_bytes=52 * 1024 * 1024,
        ),
        name="titans_qkv_proj",
    )(x2, Wqt, Wkt, Wvt)

    ntt = T // _TT

    def _cidx(b, t):
        return (b * ntt + t, 0)

    y2 = pl.pallas_call(
        _core_kernel,
        grid=(B, ntt),
        in_specs=[
            pl.BlockSpec((2, D), lambda b, t: (0, 0)),
            pl.BlockSpec((_TT, D), _cidx),
            pl.BlockSpec((_TT, D), _cidx),
            pl.BlockSpec((_TT, D), _cidx),
        ],
        out_specs=pl.BlockSpec((_TT, D), _cidx),
        out_shape=jax.ShapeDtypeStruct((B * T, D), _BF),
        scratch_shapes=[pltpu.VMEM((_NG, _DH, _GW), jnp.float32)],
        compiler_params=pltpu.CompilerParams(
            dimension_semantics=("parallel", "arbitrary"),
            vmem_limit_bytes=52 * 1024 * 1024,
        ),
        name="titans_core",
    )(ab, q2, k2, v2)

    out2 = pl.pallas_call(
        _oproj_kernel,
        grid=(nrt,),
        in_specs=[
            pl.BlockSpec((RT, D), lambda i: (i, 0)),
            pl.BlockSpec((D, D), lambda i: (0, 0)),
        ],
        out_specs=pl.BlockSpec((RT, D), lambda i: (i, 0)),
        out_shape=jax.ShapeDtypeStruct((B * T, D), jnp.float32),
        compiler_params=pltpu.CompilerParams(
            dimension_semantics=("parallel",),
            vmem_limit_bytes=52 * 1024 * 1024,
        ),
        name="titans_out_proj",
    )(y2, Wot)

    return out2.reshape(B, T, D)


def kernel(x, Wq, Wk, Wv, Wo, alpha_raw, beta_raw):
    B, T, D = x.shape
    H = _H
    xb = x.astype(_BF)
    Wqt = Wq.T.astype(_BF)
    Wkt = Wk.T.astype(_BF)
    Wvt = Wv.T.astype(_BF)
    Wot = Wo.T.astype(_BF)
    ab = jnp.repeat(
        jnp.concatenate([alpha_raw.reshape(1, H), beta_raw.reshape(1, H)],
                        axis=0), D // H, axis=1)

    devs = jax.devices()
    if len(devs) >= 2 and B % 2 == 0:
        mesh = jax.sharding.Mesh(np.array(devs[:2]), ("b",))
        P = jax.sharding.PartitionSpec
        f = shard_map(
            _impl, mesh=mesh,
            in_specs=(P("b"), P(), P(), P(), P(), P()),
            out_specs=P("b"), check_rep=False)
        return f(xb, Wqt, Wkt, Wvt, Wot, ab)
    return _impl(xb, Wqt, Wkt, Wvt, Wot, ab)
